# baseline (speedup 1.0000x reference)
import functools

import jax
import jax.numpy as jnp
from jax import lax
from jax.experimental import pallas as pl
from jax.experimental.pallas import tpu as pltpu
from jax.experimental.pallas import tpu_sc as plsc

_NW = 32
_RPW = 16384 // _NW
_R = 512
_G = 2048 // _R


def _dense_body(xa_ref, xb_ref, z_ref, xo_ref, zo_ref):
    xo_ref[0] = xa_ref[0, 0]
    xo_ref[1] = xb_ref[0, 0]
    zo_ref[0] = z_ref[0, 0]


def _dense_copies(x, z):
    out_shapes = (
        jax.ShapeDtypeStruct((2, 2048, 1024), jnp.float32),
        jax.ShapeDtypeStruct((1, 2048, 1024), jnp.float32),
    )
    in_specs = [
        pl.BlockSpec((1, 1, _R, 1024), lambda g: (2, 0, g, 0)),
        pl.BlockSpec((1, 1, _R, 1024), lambda g: (1, 1, g, 0)),
        pl.BlockSpec((1, 1, _R, 1024), lambda g: (0, 2, g, 0)),
    ]
    out_specs = (
        pl.BlockSpec((2, _R, 1024), lambda g: (0, g, 0)),
        pl.BlockSpec((1, _R, 1024), lambda g: (0, g, 0)),
    )
    return pl.pallas_call(
        _dense_body,
        grid=(_G,),
        in_specs=in_specs,
        out_specs=out_specs,
        out_shape=out_shapes,
        cost_estimate=pl.CostEstimate(
            flops=0, bytes_accessed=96 * 1024 * 1024, transcendentals=0
        ),
    )(x, x, z)


def _y_gather_body(y_hbm, out_hbm, strip_v, out_v):
    c = lax.axis_index("c")
    s = lax.axis_index("s")
    w = s * 2 + c
    pltpu.sync_copy(y_hbm.at[pl.ds(w * _RPW, _RPW), pl.ds(0, 128)], strip_v)
    lanes = lax.iota(jnp.int32, 16)
    for j in range(_RPW * 2 // 16):
        k16 = j * 16 + lanes
        out_v[j] = plsc.load_gather(strip_v, [k16 >> 1, 1 - (k16 & 1)])
    pltpu.sync_copy(out_v, out_hbm.at[w])


def _y_gather(y):
    y2 = y.reshape(16384, 2048)
    mesh = plsc.VectorSubcoreMesh(core_axis_name="c", subcore_axis_name="s")
    run = functools.partial(
        pl.kernel,
        mesh=mesh,
        out_type=jax.ShapeDtypeStruct((_NW, _RPW * 2 // 16, 16), jnp.float32),
        scratch_types=[
            pltpu.VMEM((_RPW, 128), jnp.float32),
            pltpu.VMEM((_RPW * 2 // 16, 16), jnp.float32),
        ],
        compiler_params=pltpu.CompilerParams(needs_layout_passes=False),
    )(_y_gather_body)
    out = run(y2)
    return out.reshape(4, 4096, 2)


def kernel(x, y, z):
    y_out = _y_gather(y)
    x_out, z_out = _dense_copies(x, z)
    return (x_out, y_out, z_out)
